# baseline (speedup 1.0000x reference)
import functools

import jax
import jax.numpy as jnp
import numpy as np
from jax import lax
from jax.experimental import pallas as pl
from jax.experimental.pallas import tpu as pltpu
from jax.experimental.pallas import tpu_sc as plsc

_T = 16
_T_REMAIN = 12

_SEC = np.asarray(
    jnp.sort(jax.random.choice(jax.random.key(42), _T, shape=(_T_REMAIN,), replace=False))
)


def _runs_of(sec):
    runs = []
    i = 0
    while i < len(sec):
        j = i
        while j + 1 < len(sec) and sec[j + 1] == sec[j] + 1:
            j += 1
        runs.append((int(sec[i]), i, j - i + 1))
        i = j + 1
    return runs


_RUNS = _runs_of(_SEC)

_info = plsc.get_sparse_core_info()
_NC, _NS = _info.num_cores, _info.num_subcores
_NW = _NC * _NS


def _make_sc_gather(row_elems):
    mesh = plsc.VectorSubcoreMesh(core_axis_name="c", subcore_axis_name="s")
    out_elems = _T_REMAIN * row_elems

    @functools.partial(
        pl.kernel,
        mesh=mesh,
        out_type=jax.ShapeDtypeStruct((out_elems,), jnp.float32),
        scratch_types=[pltpu.SemaphoreType.DMA] * len(_RUNS),
    )
    def sc_gather(x_hbm, out_hbm, *sems):
        wid = lax.axis_index("s") * _NC + lax.axis_index("c")
        copies = []
        for k, (src_row, dst_row, n_rows) in enumerate(_RUNS):
            plen = n_rows * row_elems // _NW
            src = src_row * row_elems + wid * plen
            dst = dst_row * row_elems + wid * plen
            c = pltpu.make_async_copy(
                x_hbm.at[pl.ds(src, plen)],
                out_hbm.at[pl.ds(dst, plen)],
                sems[k],
            )
            c.start()
            copies.append(c)
        for c in copies:
            c.wait()

    return sc_gather


def kernel(x_seq):
    T, N, C, H, W = x_seq.shape
    row = N * C * H * W
    x_flat = x_seq.reshape(T * row)
    out = _make_sc_gather(row)(x_flat)
    return out.reshape(_T_REMAIN, N, C, H, W)


# speedup vs baseline: 10.4571x; 10.4571x over previous
import functools

import jax
import jax.numpy as jnp
import numpy as np
from jax import lax
from jax.experimental import pallas as pl
from jax.experimental.pallas import tpu as pltpu
from jax.experimental.pallas import tpu_sc as plsc

_T = 16
_T_REMAIN = 12

_SEC = np.asarray(
    jnp.sort(jax.random.choice(jax.random.key(42), _T, shape=(_T_REMAIN,), replace=False))
)


def _runs_of(sec):
    runs = []
    i = 0
    while i < len(sec):
        j = i
        while j + 1 < len(sec) and sec[j + 1] == sec[j] + 1:
            j += 1
        runs.append((int(sec[i]), i, j - i + 1))
        i = j + 1
    return runs


_RUNS = _runs_of(_SEC)

_info = plsc.get_sparse_core_info()
_NC, _NS = _info.num_cores, _info.num_subcores
_NW = _NC * _NS

_CH = 32768
_NBUF = 3
_LOOKAHEAD = 2


def _make_sc_gather(row_elems):
    mesh = plsc.VectorSubcoreMesh(core_axis_name="c", subcore_axis_name="s")
    out_elems = _T_REMAIN * row_elems

    chunks = []
    for src_row, dst_row, n_rows in _RUNS:
        plen = n_rows * row_elems // _NW
        assert plen % _CH == 0
        prows = plen // _CH
        for j in range(prows):
            chunks.append((src_row * row_elems // _CH + j,
                           dst_row * row_elems // _CH + j,
                           prows))
    n = len(chunks)

    @functools.partial(
        pl.kernel,
        mesh=mesh,
        out_type=jax.ShapeDtypeStruct((out_elems // _CH, _CH), jnp.float32),
        scratch_types=[pltpu.VMEM_SHARED((_NS * _NBUF, _CH), jnp.float32)]
        + [pltpu.SemaphoreType.DMA] * (2 * _NBUF),
    )
    def sc_gather(x_hbm, out_hbm, shared, *sems):
        sin, sout = sems[:_NBUF], sems[_NBUF:]
        sid = lax.axis_index("s")
        wid = sid * _NC + lax.axis_index("c")

        def in_copy(j):
            src, _, prows = chunks[j]
            return pltpu.make_async_copy(
                x_hbm.at[pl.ds(src + wid * prows, 1)],
                shared.at[pl.ds(sid * _NBUF + j % _NBUF, 1)],
                sin[j % _NBUF])

        def out_copy(j):
            _, dst, prows = chunks[j]
            return pltpu.make_async_copy(
                shared.at[pl.ds(sid * _NBUF + j % _NBUF, 1)],
                out_hbm.at[pl.ds(dst + wid * prows, 1)],
                sout[j % _NBUF])

        for j in range(min(_LOOKAHEAD, n)):
            in_copy(j).start()
        for j in range(n):
            jj = j + _LOOKAHEAD
            if jj < n:
                if jj >= _NBUF:
                    out_copy(jj - _NBUF).wait()
                in_copy(jj).start()
            in_copy(j).wait()
            out_copy(j).start()
        for j in range(max(n - _NBUF, 0), n):
            out_copy(j).wait()

    return sc_gather


def kernel(x_seq):
    T, N, C, H, W = x_seq.shape
    row = N * C * H * W
    x2 = x_seq.reshape(T * row // _CH, _CH)
    out = _make_sc_gather(row)(x2)
    return out.reshape(_T_REMAIN, N, C, H, W)


# speedup vs baseline: 34.2841x; 3.2786x over previous
import functools

import jax
import jax.numpy as jnp
import numpy as np
from jax import lax
from jax.experimental import pallas as pl
from jax.experimental.pallas import tpu as pltpu
from jax.experimental.pallas import tpu_sc as plsc

_T = 16
_T_REMAIN = 12

_SEC = np.asarray(
    jnp.sort(jax.random.choice(jax.random.key(42), _T, shape=(_T_REMAIN,), replace=False))
)


def _runs_of(sec):
    runs = []
    i = 0
    while i < len(sec):
        j = i
        while j + 1 < len(sec) and sec[j + 1] == sec[j] + 1:
            j += 1
        runs.append((int(sec[i]), i, j - i + 1))
        i = j + 1
    return runs


_RUNS = _runs_of(_SEC)

_info = plsc.get_sparse_core_info()
_NC, _NS = _info.num_cores, _info.num_subcores
_NW = _NC * _NS

_CH = 32768
_NBUF = 3
_LOOKAHEAD = 2


def _make_sc_gather(row_elems):
    mesh = plsc.VectorSubcoreMesh(core_axis_name="c", subcore_axis_name="s")
    out_elems = _T_REMAIN * row_elems

    chunks = []
    for src_row, dst_row, n_rows in _RUNS:
        plen = n_rows * row_elems // _NW
        assert plen % _CH == 0
        for j in range(plen // _CH):
            chunks.append((src_row * row_elems + j * _CH,
                           dst_row * row_elems + j * _CH,
                           plen))
    n = len(chunks)

    @functools.partial(
        pl.kernel,
        mesh=mesh,
        out_type=jax.ShapeDtypeStruct((out_elems,), jnp.float32),
        scratch_types=[pltpu.VMEM((_CH,), jnp.float32)] * _NBUF
        + [pltpu.SemaphoreType.DMA] * (2 * _NBUF),
    )
    def sc_gather(x_hbm, out_hbm, *scratch):
        bufs = scratch[:_NBUF]
        sin, sout = scratch[_NBUF:2 * _NBUF], scratch[2 * _NBUF:]
        wid = lax.axis_index("s") * _NC + lax.axis_index("c")

        def in_copy(j):
            src, _, plen = chunks[j]
            return pltpu.make_async_copy(
                x_hbm.at[pl.ds(src + wid * plen, _CH)],
                bufs[j % _NBUF], sin[j % _NBUF])

        def out_copy(j):
            _, dst, plen = chunks[j]
            return pltpu.make_async_copy(
                bufs[j % _NBUF],
                out_hbm.at[pl.ds(dst + wid * plen, _CH)], sout[j % _NBUF])

        for j in range(min(_LOOKAHEAD, n)):
            in_copy(j).start()
        for j in range(n):
            jj = j + _LOOKAHEAD
            if jj < n:
                if jj >= _NBUF:
                    out_copy(jj - _NBUF).wait()
                in_copy(jj).start()
            in_copy(j).wait()
            out_copy(j).start()
        for j in range(max(n - _NBUF, 0), n):
            out_copy(j).wait()

    return sc_gather


def kernel(x_seq):
    T, N, C, H, W = x_seq.shape
    row = N * C * H * W
    x_flat = x_seq.reshape(T * row)
    out = _make_sc_gather(row)(x_flat)
    return out.reshape(_T_REMAIN, N, C, H, W)


# speedup vs baseline: 34.3136x; 1.0009x over previous
import functools

import jax
import jax.numpy as jnp
import numpy as np
from jax import lax
from jax.experimental import pallas as pl
from jax.experimental.pallas import tpu as pltpu
from jax.experimental.pallas import tpu_sc as plsc

_T = 16
_T_REMAIN = 12

_SEC = np.asarray(
    jnp.sort(jax.random.choice(jax.random.key(42), _T, shape=(_T_REMAIN,), replace=False))
)


def _runs_of(sec):
    runs = []
    i = 0
    while i < len(sec):
        j = i
        while j + 1 < len(sec) and sec[j + 1] == sec[j] + 1:
            j += 1
        runs.append((int(sec[i]), i, j - i + 1))
        i = j + 1
    return runs


_RUNS = _runs_of(_SEC)

_info = plsc.get_sparse_core_info()
_NC, _NS = _info.num_cores, _info.num_subcores
_NW = _NC * _NS

_CH = 32768
_NBUF = 3
_LOOKAHEAD = 2
_NBUF_B = 3
_CH_B = 8192
_N_B = 3


def _make_sc_gather(row_elems):
    mesh = plsc.VectorSubcoreMesh(core_axis_name="c", subcore_axis_name="s")
    out_elems = _T_REMAIN * row_elems

    chunks = []
    for src_row, dst_row, n_rows in _RUNS:
        plen = n_rows * row_elems // _NW
        assert plen % _CH == 0
        for j in range(plen // _CH):
            chunks.append((src_row * row_elems + j * _CH,
                           dst_row * row_elems + j * _CH,
                           plen))
    chunks_b = []
    for src, dst, plen in chunks[:_N_B]:
        for j in range(_CH // _CH_B):
            chunks_b.append((src + j * _CH_B, dst + j * _CH_B, plen))
    chunks_a = chunks[_N_B:]
    na, nb = len(chunks_a), len(chunks_b)

    @functools.partial(
        pl.kernel,
        mesh=mesh,
        out_type=jax.ShapeDtypeStruct((out_elems,), jnp.float32),
        scratch_types=[pltpu.VMEM((_CH,), jnp.float32)] * _NBUF
        + [pltpu.VMEM_SHARED((_NS * _NBUF_B * _CH_B,), jnp.float32)]
        + [pltpu.SemaphoreType.DMA] * (2 * _NBUF + 2 * _NBUF_B),
    )
    def sc_gather(x_hbm, out_hbm, *scratch):
        bufs = scratch[:_NBUF]
        shared = scratch[_NBUF]
        sems = scratch[_NBUF + 1:]
        sin, sout = sems[:_NBUF], sems[_NBUF:2 * _NBUF]
        sin_b = sems[2 * _NBUF:2 * _NBUF + _NBUF_B]
        sout_b = sems[2 * _NBUF + _NBUF_B:]
        sid = lax.axis_index("s")
        wid = sid * _NC + lax.axis_index("c")

        def in_copy(j):
            src, _, plen = chunks_a[j]
            return pltpu.make_async_copy(
                x_hbm.at[pl.ds(src + wid * plen, _CH)],
                bufs[j % _NBUF], sin[j % _NBUF])

        def out_copy(j):
            _, dst, plen = chunks_a[j]
            return pltpu.make_async_copy(
                bufs[j % _NBUF],
                out_hbm.at[pl.ds(dst + wid * plen, _CH)], sout[j % _NBUF])

        def b_slot(k):
            return (sid * _NBUF_B + k % _NBUF_B) * _CH_B

        def in_copy_b(k):
            src, _, plen = chunks_b[k]
            return pltpu.make_async_copy(
                x_hbm.at[pl.ds(src + wid * plen, _CH_B)],
                shared.at[pl.ds(b_slot(k), _CH_B)], sin_b[k % _NBUF_B])

        def out_copy_b(k):
            _, dst, plen = chunks_b[k]
            return pltpu.make_async_copy(
                shared.at[pl.ds(b_slot(k), _CH_B)],
                out_hbm.at[pl.ds(dst + wid * plen, _CH_B)], sout_b[k % _NBUF_B])

        stride = max(na // max(nb, 1), 1)

        def b_step(k):
            kk = k + 1
            if kk < nb:
                if kk >= _NBUF_B:
                    out_copy_b(kk - _NBUF_B).wait()
                in_copy_b(kk).start()
            in_copy_b(k).wait()
            out_copy_b(k).start()

        if nb:
            in_copy_b(0).start()
        for j in range(min(_LOOKAHEAD, na)):
            in_copy(j).start()
        for j in range(na):
            jj = j + _LOOKAHEAD
            if jj < na:
                if jj >= _NBUF:
                    out_copy(jj - _NBUF).wait()
                in_copy(jj).start()
            in_copy(j).wait()
            out_copy(j).start()
            if j % stride == 0 and j // stride < nb:
                b_step(j // stride)
        for k in range(na // stride if na // stride < nb else nb, nb):
            b_step(k)
        for j in range(max(na - _NBUF, 0), na):
            out_copy(j).wait()
        for k in range(max(nb - _NBUF_B, 0), nb):
            out_copy_b(k).wait()

    return sc_gather


def kernel(x_seq):
    T, N, C, H, W = x_seq.shape
    row = N * C * H * W
    x_flat = x_seq.reshape(T * row)
    out = _make_sc_gather(row)(x_flat)
    return out.reshape(_T_REMAIN, N, C, H, W)
